# baseline (speedup 1.0000x reference)
import functools

import jax
import jax.numpy as jnp
from jax import lax
from jax.experimental import pallas as pl
from jax.experimental.pallas import tpu as pltpu
from jax.experimental.pallas import tpu_sc as plsc

_K = 16
_HW = 1024
_NPAIR = 32


def _proj_body(x_ref, w_ref, b1_ref, b2_ref, a_ref, b_ref):
    for f in range(9):
        Xf = x_ref[0, :, f * _HW:(f + 1) * _HW]
        y = jnp.dot(w_ref[...], Xf,
                    preferred_element_type=jnp.float32)
        if f < 8:
            a_ref[f, :] = y[0, :] + b1_ref[0]
        if f >= 1:
            b_ref[f - 1, :] = y[1, :] + b2_ref[0]


def _z_body(a_ref, b_ref, z_ref):
    def pair(p, carry):
        av = a_ref[pl.ds(p, 1), :]
        bv = b_ref[pl.ds(p, 1), :]
        at = jnp.transpose(av)
        bmax = jnp.max(bv)
        bmin = jnp.min(bv)
        m = jnp.where(at > 0, at * bmax, at * bmin)
        E = jnp.exp(at * bv - m)
        z_ref[pl.ds(p, 1)] = jnp.sum(
            E, axis=1, keepdims=True).reshape(1, _HW, 1)
        return carry

    lax.fori_loop(0, _NPAIR, pair, 0)


def _sc_body(a_hbm, b_hbm, z_hbm, out_hbm,
             a_v, b_v, z_v, m_v, r_v, top_v, bot_v, vals_v):
    cid = lax.axis_index("c")
    sid = lax.axis_index("s")
    p = sid * 2 + cid
    n = p // 8
    t = p % 8

    pltpu.sync_copy(a_hbm.at[p], a_v)
    pltpu.sync_copy(b_hbm.at[p], b_v)
    pltpu.sync_copy(z_hbm.at[p], z_v)

    def merge_body(i, carry):
        top, bot = carry
        cha = jnp.sort(b_v[pl.ds(i * _K, _K)])
        chd = jnp.flip(cha, 0)
        top = jnp.flip(jnp.sort(jnp.maximum(top, cha)), 0)
        bot = jnp.sort(jnp.minimum(bot, chd))
        return top, bot

    top, bot = lax.fori_loop(
        0, _HW // _K, merge_body,
        (jnp.full((_K,), -jnp.inf, jnp.float32),
         jnp.full((_K,), jnp.inf, jnp.float32)))
    top_v[...] = top
    bot_v[...] = bot

    bmax = jnp.broadcast_to(jnp.max(top), (_K,))
    bmin = jnp.broadcast_to(jnp.min(bot), (_K,))

    def mr_body(i, carry):
        ach = a_v[pl.ds(i * _K, _K)]
        m_v[pl.ds(i * _K, _K)] = jnp.where(ach > 0, ach * bmax, ach * bmin)
        r_v[pl.ds(i * _K, _K)] = 1.0 / z_v[pl.ds(i * _K, _K)]
        return carry

    lax.fori_loop(0, _HW // _K, mr_body, 0)

    tb = [bmax] + [plsc.load_gather(top_v, [jnp.full((_K,), k, jnp.int32)])
                   for k in range(1, _K)]
    bb = [bmin] + [plsc.load_gather(bot_v, [jnp.full((_K,), k, jnp.int32)])
                   for k in range(1, _K)]
    lane = lax.iota(jnp.int32, _K)

    def val_body(i, carry):
        ach = a_v[pl.ds(i * _K, _K)]
        mch = m_v[pl.ds(i * _K, _K)]
        rch = r_v[pl.ds(i * _K, _K)]
        pos = ach > 0
        base = (i * _K + lane) * _K
        for k in range(_K):
            csel = jnp.where(pos, tb[k], bb[k])
            val = jnp.exp(ach * csel - mch) * rch
            plsc.store_scatter(vals_v, [base + k], val)
        return carry

    lax.fori_loop(0, _HW // _K, val_body, 0)

    for kk in range(_K):
        row = n * (_K * 8) + kk * 8 + t
        pltpu.sync_copy(vals_v.at[pl.ds(kk * _HW, _HW)], out_hbm.at[row])


@functools.cache
def _sc_topk():
    mesh = plsc.VectorSubcoreMesh(core_axis_name="c", subcore_axis_name="s")
    return pl.kernel(
        _sc_body,
        mesh=mesh,
        compiler_params=pltpu.CompilerParams(needs_layout_passes=False),
        out_type=jax.ShapeDtypeStruct((4 * _K * 8, _HW), jnp.float32),
        scratch_types=[
            pltpu.VMEM((_HW,), jnp.float32),
            pltpu.VMEM((_HW,), jnp.float32),
            pltpu.VMEM((_HW,), jnp.float32),
            pltpu.VMEM((_HW,), jnp.float32),
            pltpu.VMEM((_HW,), jnp.float32),
            pltpu.VMEM((_K,), jnp.float32),
            pltpu.VMEM((_K,), jnp.float32),
            pltpu.VMEM((_HW * _K,), jnp.float32),
        ],
    )


def kernel(x, W1, b1, W2, b2):
    n, c, s, h, w = x.shape
    hw = h * w
    xr = x.reshape(n, c, s * hw)
    Wc = jnp.concatenate([W1, W2], axis=0)

    npair = n * (s - 1)
    a_full, b_full = pl.pallas_call(
        _proj_body,
        grid=(n,),
        in_specs=[
            pl.BlockSpec((1, c, s * hw), lambda i: (i, 0, 0)),
            pl.BlockSpec((2, c), lambda i: (0, 0)),
            pl.BlockSpec((1,), lambda i: (0,)),
            pl.BlockSpec((1,), lambda i: (0,)),
        ],
        out_specs=[
            pl.BlockSpec((s - 1, hw), lambda i: (i, 0)),
            pl.BlockSpec((s - 1, hw), lambda i: (i, 0)),
        ],
        out_shape=[jax.ShapeDtypeStruct((npair, hw), jnp.float32)] * 2,
    )(xr, Wc, b1, b2)

    Z = pl.pallas_call(
        _z_body,
        grid=(1,),
        in_specs=[
            pl.BlockSpec((npair, hw), lambda i: (0, 0)),
            pl.BlockSpec((npair, hw), lambda i: (0, 0)),
        ],
        out_specs=pl.BlockSpec((npair, hw, 1), lambda i: (0, 0, 0)),
        out_shape=jax.ShapeDtypeStruct((npair, hw, 1), jnp.float32),
    )(a_full, b_full)
    z_full = Z.reshape(npair, hw)

    out = _sc_topk()(a_full, b_full, z_full)
    return out.reshape(n, _K, s - 1, h, w)


# speedup vs baseline: 1.4123x; 1.4123x over previous
import functools

import jax
import jax.numpy as jnp
from jax import lax
from jax.experimental import pallas as pl
from jax.experimental.pallas import tpu as pltpu
from jax.experimental.pallas import tpu_sc as plsc

_K = 16
_HW = 1024
_NPAIR = 32


def _proj_body(x_ref, w_ref, b1_ref, b2_ref, a_ref, b_ref):
    for f in range(9):
        Xf = x_ref[0, f].reshape(_HW, x_ref.shape[-1])
        y = jnp.dot(Xf, w_ref[...],
                    preferred_element_type=jnp.float32)
        yt = jnp.transpose(y)
        if f < 8:
            a_ref[f, :] = yt[0, :] + b1_ref[0]
        if f >= 1:
            b_ref[f - 1, :] = yt[1, :] + b2_ref[0]


def _z_body(a_ref, b_ref, z_ref):
    def pair(p, carry):
        av = a_ref[pl.ds(p, 1), :]
        bv = b_ref[pl.ds(p, 1), :]
        at = jnp.transpose(av)
        bmax = jnp.max(bv)
        bmin = jnp.min(bv)
        m = jnp.where(at > 0, at * bmax, at * bmin)
        E = jnp.exp(at * bv - m)
        zc = jnp.sum(E, axis=1, keepdims=True)
        z_ref[pl.ds(p, 1), :] = jnp.transpose(zc)
        return carry

    lax.fori_loop(0, _NPAIR, pair, 0)


def _sc_body(a_hbm, b_hbm, z_hbm, out_hbm,
             a_v, b_v, z_v, m_v, r_v, top_v, bot_v, vals_v):
    cid = lax.axis_index("c")
    sid = lax.axis_index("s")
    p = sid * 2 + cid
    n = p // 8
    t = p % 8

    pltpu.sync_copy(a_hbm.at[p], a_v)
    pltpu.sync_copy(b_hbm.at[p], b_v)
    pltpu.sync_copy(z_hbm.at[p], z_v)

    def merge_body(i, carry):
        top, bot = carry
        cha = jnp.sort(b_v[pl.ds(i * _K, _K)])
        chd = jnp.flip(cha, 0)
        top = jnp.flip(jnp.sort(jnp.maximum(top, cha)), 0)
        bot = jnp.sort(jnp.minimum(bot, chd))
        return top, bot

    top, bot = lax.fori_loop(
        0, _HW // _K, merge_body,
        (jnp.full((_K,), -jnp.inf, jnp.float32),
         jnp.full((_K,), jnp.inf, jnp.float32)))
    top_v[...] = top
    bot_v[...] = bot

    bmax = jnp.broadcast_to(jnp.max(top), (_K,))
    bmin = jnp.broadcast_to(jnp.min(bot), (_K,))

    def mr_body(i, carry):
        ach = a_v[pl.ds(i * _K, _K)]
        m_v[pl.ds(i * _K, _K)] = jnp.where(ach > 0, ach * bmax, ach * bmin)
        r_v[pl.ds(i * _K, _K)] = 1.0 / z_v[pl.ds(i * _K, _K)]
        return carry

    lax.fori_loop(0, _HW // _K, mr_body, 0)

    tb = [bmax] + [plsc.load_gather(top_v, [jnp.full((_K,), k, jnp.int32)])
                   for k in range(1, _K)]
    bb = [bmin] + [plsc.load_gather(bot_v, [jnp.full((_K,), k, jnp.int32)])
                   for k in range(1, _K)]
    lane = lax.iota(jnp.int32, _K)

    def val_body(i, carry):
        ach = a_v[pl.ds(i * _K, _K)]
        mch = m_v[pl.ds(i * _K, _K)]
        rch = r_v[pl.ds(i * _K, _K)]
        pos = ach > 0
        base = (i * _K + lane) * _K
        for k in range(_K):
            csel = jnp.where(pos, tb[k], bb[k])
            val = jnp.exp(ach * csel - mch) * rch
            plsc.store_scatter(vals_v, [base + k], val)
        return carry

    lax.fori_loop(0, _HW // _K, val_body, 0)

    for kk in range(_K):
        row = n * (_K * 8) + kk * 8 + t
        pltpu.sync_copy(vals_v.at[pl.ds(kk * _HW, _HW)], out_hbm.at[row])


@functools.cache
def _sc_topk():
    mesh = plsc.VectorSubcoreMesh(core_axis_name="c", subcore_axis_name="s")
    return pl.kernel(
        _sc_body,
        mesh=mesh,
        compiler_params=pltpu.CompilerParams(needs_layout_passes=False),
        out_type=jax.ShapeDtypeStruct((4 * _K * 8, _HW), jnp.float32),
        scratch_types=[
            pltpu.VMEM((_HW,), jnp.float32),
            pltpu.VMEM((_HW,), jnp.float32),
            pltpu.VMEM((_HW,), jnp.float32),
            pltpu.VMEM((_HW,), jnp.float32),
            pltpu.VMEM((_HW,), jnp.float32),
            pltpu.VMEM((_K,), jnp.float32),
            pltpu.VMEM((_K,), jnp.float32),
            pltpu.VMEM((_HW * _K,), jnp.float32),
        ],
    )


def kernel(x, W1, b1, W2, b2):
    n, c, s, h, w = x.shape
    hw = h * w
    xt = jnp.transpose(x, (0, 2, 3, 4, 1))
    WcT = jnp.concatenate([W1, W2], axis=0).T

    npair = n * (s - 1)
    a_full, b_full = pl.pallas_call(
        _proj_body,
        grid=(n,),
        in_specs=[
            pl.BlockSpec((1, s, h, w, c), lambda i: (i, 0, 0, 0, 0)),
            pl.BlockSpec((c, 2), lambda i: (0, 0)),
            pl.BlockSpec((1,), lambda i: (0,)),
            pl.BlockSpec((1,), lambda i: (0,)),
        ],
        out_specs=[
            pl.BlockSpec((s - 1, hw), lambda i: (i, 0)),
            pl.BlockSpec((s - 1, hw), lambda i: (i, 0)),
        ],
        out_shape=[jax.ShapeDtypeStruct((npair, hw), jnp.float32)] * 2,
    )(xt, WcT, b1, b2)

    z_full = pl.pallas_call(
        _z_body,
        grid=(1,),
        in_specs=[
            pl.BlockSpec((npair, hw), lambda i: (0, 0)),
            pl.BlockSpec((npair, hw), lambda i: (0, 0)),
        ],
        out_specs=pl.BlockSpec((npair, hw), lambda i: (0, 0)),
        out_shape=jax.ShapeDtypeStruct((npair, hw), jnp.float32),
    )(a_full, b_full)

    out = _sc_topk()(a_full, b_full, z_full)
    return out.reshape(n, _K, s - 1, h, w)


# speedup vs baseline: 1.5069x; 1.0670x over previous
import functools

import jax
import jax.numpy as jnp
from jax import lax
from jax.experimental import pallas as pl
from jax.experimental.pallas import tpu as pltpu
from jax.experimental.pallas import tpu_sc as plsc

_K = 16
_HW = 1024
_NPAIR = 32


def _proj_z_body(x_ref, w_ref, b1_ref, b2_ref, a_ref, b_ref, z_ref):
    for f in range(9):
        Xf = x_ref[0, f].reshape(_HW, x_ref.shape[-1])
        y = jnp.dot(Xf, w_ref[...],
                    preferred_element_type=jnp.float32)
        yt = jnp.transpose(y)
        if f < 8:
            a_ref[f, :] = yt[0, :] + b1_ref[0]
        if f >= 1:
            b_ref[f - 1, :] = yt[1, :] + b2_ref[0]

    def pair(p, carry):
        av = a_ref[pl.ds(p, 1), :]
        bv = b_ref[pl.ds(p, 1), :]
        at = jnp.transpose(av)
        bmax = jnp.max(bv)
        bmin = jnp.min(bv)
        m = jnp.where(at > 0, at * bmax, at * bmin)
        E = jnp.exp(at * bv - m)
        zc = jnp.sum(E, axis=1, keepdims=True)
        z_ref[pl.ds(p, 1), :] = jnp.transpose(zc)
        return carry

    lax.fori_loop(0, 8, pair, 0)


def _sc_body(a_hbm, b_hbm, z_hbm, out_hbm,
             a_v, b_v, z_v, m_v, r_v, top_v, bot_v, vals_v):
    cid = lax.axis_index("c")
    sid = lax.axis_index("s")
    p = sid * 2 + cid
    n = p // 8
    t = p % 8

    pltpu.sync_copy(a_hbm.at[p], a_v)
    pltpu.sync_copy(b_hbm.at[p], b_v)
    pltpu.sync_copy(z_hbm.at[p], z_v)

    def merge_body(i, carry):
        top, bot = carry
        cha = jnp.sort(b_v[pl.ds(i * _K, _K)])
        chd = jnp.flip(cha, 0)
        top = jnp.flip(jnp.sort(jnp.maximum(top, cha)), 0)
        bot = jnp.sort(jnp.minimum(bot, chd))
        return top, bot

    top, bot = lax.fori_loop(
        0, _HW // _K, merge_body,
        (jnp.full((_K,), -jnp.inf, jnp.float32),
         jnp.full((_K,), jnp.inf, jnp.float32)))
    top_v[...] = top
    bot_v[...] = bot

    bmax = jnp.broadcast_to(jnp.max(top), (_K,))
    bmin = jnp.broadcast_to(jnp.min(bot), (_K,))

    def mr_body(i, carry):
        ach = a_v[pl.ds(i * _K, _K)]
        m_v[pl.ds(i * _K, _K)] = jnp.where(ach > 0, ach * bmax, ach * bmin)
        r_v[pl.ds(i * _K, _K)] = 1.0 / z_v[pl.ds(i * _K, _K)]
        return carry

    lax.fori_loop(0, _HW // _K, mr_body, 0)

    tb = [bmax] + [plsc.load_gather(top_v, [jnp.full((_K,), k, jnp.int32)])
                   for k in range(1, _K)]
    bb = [bmin] + [plsc.load_gather(bot_v, [jnp.full((_K,), k, jnp.int32)])
                   for k in range(1, _K)]
    lane = lax.iota(jnp.int32, _K)

    def val_body(i, carry):
        ach = a_v[pl.ds(i * _K, _K)]
        mch = m_v[pl.ds(i * _K, _K)]
        rch = r_v[pl.ds(i * _K, _K)]
        pos = ach > 0
        base = (i * _K + lane) * _K
        for k in range(_K):
            csel = jnp.where(pos, tb[k], bb[k])
            val = jnp.exp(ach * csel - mch) * rch
            plsc.store_scatter(vals_v, [base + k], val)
        return carry

    lax.fori_loop(0, _HW // _K, val_body, 0)

    for kk in range(_K):
        row = n * (_K * 8) + kk * 8 + t
        pltpu.sync_copy(vals_v.at[pl.ds(kk * _HW, _HW)], out_hbm.at[row])


@functools.cache
def _sc_topk():
    mesh = plsc.VectorSubcoreMesh(core_axis_name="c", subcore_axis_name="s")
    return pl.kernel(
        _sc_body,
        mesh=mesh,
        compiler_params=pltpu.CompilerParams(needs_layout_passes=False),
        out_type=jax.ShapeDtypeStruct((4 * _K * 8, _HW), jnp.float32),
        scratch_types=[
            pltpu.VMEM((_HW,), jnp.float32),
            pltpu.VMEM((_HW,), jnp.float32),
            pltpu.VMEM((_HW,), jnp.float32),
            pltpu.VMEM((_HW,), jnp.float32),
            pltpu.VMEM((_HW,), jnp.float32),
            pltpu.VMEM((_K,), jnp.float32),
            pltpu.VMEM((_K,), jnp.float32),
            pltpu.VMEM((_HW * _K,), jnp.float32),
        ],
    )


def kernel(x, W1, b1, W2, b2):
    n, c, s, h, w = x.shape
    hw = h * w
    xt = jnp.transpose(x, (0, 2, 3, 4, 1))
    WcT = jnp.concatenate([W1, W2], axis=0).T

    npair = n * (s - 1)
    a_full, b_full, z_full = pl.pallas_call(
        _proj_z_body,
        grid=(n,),
        in_specs=[
            pl.BlockSpec((1, s, h, w, c), lambda i: (i, 0, 0, 0, 0)),
            pl.BlockSpec((c, 2), lambda i: (0, 0)),
            pl.BlockSpec((1,), lambda i: (0,)),
            pl.BlockSpec((1,), lambda i: (0,)),
        ],
        out_specs=[
            pl.BlockSpec((s - 1, hw), lambda i: (i, 0)),
            pl.BlockSpec((s - 1, hw), lambda i: (i, 0)),
            pl.BlockSpec((s - 1, hw), lambda i: (i, 0)),
        ],
        out_shape=[jax.ShapeDtypeStruct((npair, hw), jnp.float32)] * 3,
    )(xt, WcT, b1, b2)

    out = _sc_topk()(a_full, b_full, z_full)
    return out.reshape(n, _K, s - 1, h, w)


# speedup vs baseline: 1.5467x; 1.0264x over previous
import functools

import jax
import jax.numpy as jnp
from jax import lax
from jax.experimental import pallas as pl
from jax.experimental.pallas import tpu as pltpu
from jax.experimental.pallas import tpu_sc as plsc

_K = 16
_HW = 1024
_NPAIR = 32


def _proj_z_body(x_ref, w_ref, b1_ref, b2_ref, a_ref, b_ref, z_ref):
    for f in range(9):
        Xf = x_ref[0, f].reshape(_HW, x_ref.shape[-1])
        y = jnp.dot(Xf, w_ref[...],
                    preferred_element_type=jnp.float32)
        yt = jnp.transpose(y)
        if f < 8:
            a_ref[f, :] = yt[0, :] + b1_ref[0]
        if f >= 1:
            b_ref[f - 1, :] = yt[1, :] + b2_ref[0]

    for p in range(8):
        av = a_ref[p:p + 1, :]
        bv = b_ref[p:p + 1, :]
        at = jnp.transpose(av)
        bmax = jnp.max(bv)
        bmin = jnp.min(bv)
        m = jnp.where(at > 0, at * bmax, at * bmin)
        E = jnp.exp(at * bv - m)
        zc = jnp.sum(E, axis=1, keepdims=True)
        z_ref[p:p + 1, :] = jnp.transpose(zc)


def _sc_body(a_hbm, b_hbm, z_hbm, out_hbm,
             a_v, b_v, z_v, m_v, r_v, top_v, bot_v, vals_v):
    cid = lax.axis_index("c")
    sid = lax.axis_index("s")
    p = sid * 2 + cid
    n = p // 8
    t = p % 8

    pltpu.sync_copy(a_hbm.at[p], a_v)
    pltpu.sync_copy(b_hbm.at[p], b_v)
    pltpu.sync_copy(z_hbm.at[p], z_v)

    def merge_body(i, carry):
        top, bot = carry
        cha = jnp.sort(b_v[pl.ds(i * _K, _K)])
        chd = jnp.flip(cha, 0)
        top = jnp.flip(jnp.sort(jnp.maximum(top, cha)), 0)
        bot = jnp.sort(jnp.minimum(bot, chd))
        return top, bot

    top, bot = lax.fori_loop(
        0, _HW // _K, merge_body,
        (jnp.full((_K,), -jnp.inf, jnp.float32),
         jnp.full((_K,), jnp.inf, jnp.float32)))
    top_v[...] = top
    bot_v[...] = bot

    bmax = jnp.broadcast_to(jnp.max(top), (_K,))
    bmin = jnp.broadcast_to(jnp.min(bot), (_K,))

    def mr_body(i, carry):
        ach = a_v[pl.ds(i * _K, _K)]
        m_v[pl.ds(i * _K, _K)] = jnp.where(ach > 0, ach * bmax, ach * bmin)
        r_v[pl.ds(i * _K, _K)] = 1.0 / z_v[pl.ds(i * _K, _K)]
        return carry

    lax.fori_loop(0, _HW // _K, mr_body, 0)

    tb = [bmax] + [plsc.load_gather(top_v, [jnp.full((_K,), k, jnp.int32)])
                   for k in range(1, _K)]
    bb = [bmin] + [plsc.load_gather(bot_v, [jnp.full((_K,), k, jnp.int32)])
                   for k in range(1, _K)]
    lane = lax.iota(jnp.int32, _K)

    def val_body(i, carry):
        ach = a_v[pl.ds(i * _K, _K)]
        mch = m_v[pl.ds(i * _K, _K)]
        rch = r_v[pl.ds(i * _K, _K)]
        pos = ach > 0
        base = (i * _K + lane) * _K
        for k in range(_K):
            csel = jnp.where(pos, tb[k], bb[k])
            val = jnp.exp(ach * csel - mch) * rch
            plsc.store_scatter(vals_v, [base + k], val)
        return carry

    lax.fori_loop(0, _HW // _K, val_body, 0)

    for kk in range(_K):
        row = n * (_K * 8) + kk * 8 + t
        pltpu.sync_copy(vals_v.at[pl.ds(kk * _HW, _HW)], out_hbm.at[row])


@functools.cache
def _sc_topk():
    mesh = plsc.VectorSubcoreMesh(core_axis_name="c", subcore_axis_name="s")
    return pl.kernel(
        _sc_body,
        mesh=mesh,
        compiler_params=pltpu.CompilerParams(needs_layout_passes=False),
        out_type=jax.ShapeDtypeStruct((4 * _K * 8, _HW), jnp.float32),
        scratch_types=[
            pltpu.VMEM((_HW,), jnp.float32),
            pltpu.VMEM((_HW,), jnp.float32),
            pltpu.VMEM((_HW,), jnp.float32),
            pltpu.VMEM((_HW,), jnp.float32),
            pltpu.VMEM((_HW,), jnp.float32),
            pltpu.VMEM((_K,), jnp.float32),
            pltpu.VMEM((_K,), jnp.float32),
            pltpu.VMEM((_HW * _K,), jnp.float32),
        ],
    )


def kernel(x, W1, b1, W2, b2):
    n, c, s, h, w = x.shape
    hw = h * w
    xt = jnp.transpose(x, (0, 2, 3, 4, 1))
    WcT = jnp.concatenate([W1, W2], axis=0).T

    npair = n * (s - 1)
    a_full, b_full, z_full = pl.pallas_call(
        _proj_z_body,
        grid=(n,),
        in_specs=[
            pl.BlockSpec((1, s, h, w, c), lambda i: (i, 0, 0, 0, 0)),
            pl.BlockSpec((c, 2), lambda i: (0, 0)),
            pl.BlockSpec((1,), lambda i: (0,)),
            pl.BlockSpec((1,), lambda i: (0,)),
        ],
        out_specs=[
            pl.BlockSpec((s - 1, hw), lambda i: (i, 0)),
            pl.BlockSpec((s - 1, hw), lambda i: (i, 0)),
            pl.BlockSpec((s - 1, hw), lambda i: (i, 0)),
        ],
        out_shape=[jax.ShapeDtypeStruct((npair, hw), jnp.float32)] * 3,
    )(xt, WcT, b1, b2)

    out = _sc_topk()(a_full, b_full, z_full)
    return out.reshape(n, _K, s - 1, h, w)
